# Initial kernel scaffold; baseline (speedup 1.0000x reference)
import jax
import jax.numpy as jnp
from jax.experimental import pallas as pl


def kernel(x, lut):
    raise NotImplementedError("write your pallas kernel here")


# speedup vs baseline: 4.5639x; 4.5639x over previous
import functools
import math

import jax
import jax.numpy as jnp
from jax import lax
from jax.experimental import pallas as pl
from jax.experimental.pallas import tpu as pltpu
from jax.experimental.pallas import tpu_sc as plsc

D = 32
L = 16
SCALE = math.sqrt(D)
NC, NS = 2, 16
NW = NC * NS
K = 1024
U = 8


def _emb_body(n_chunks, x_hbm, lut_hbm, out_hbm, idx_v, rows_v, sem):
    wid = lax.axis_index("s") * NC + lax.axis_index("c")
    per_w = n_chunks * K
    base = wid * per_w

    def chunk_body(ci, carry):
        off = base + ci * K
        pltpu.sync_copy(x_hbm.at[pl.ds(off, K)], idx_v)
        pltpu.async_copy(lut_hbm.at[idx_v], rows_v, sem).wait()

        def row_body(r, c2):
            r0 = r * U
            for u in range(U):
                for h in range(D // L):
                    sl = pl.ds(h * L, L)
                    rows_v[r0 + u, sl] = rows_v[r0 + u, sl] * SCALE
            return c2

        lax.fori_loop(0, K // U, row_body, 0)
        pltpu.sync_copy(rows_v, out_hbm.at[pl.ds(off, K)])
        return carry

    lax.fori_loop(0, n_chunks, chunk_body, 0)


def kernel(x, lut):
    B, C = x.shape
    N = B * C
    assert N % (NW * K) == 0
    n_chunks = N // (NW * K)
    xf = x.reshape(N)

    mesh = plsc.VectorSubcoreMesh(core_axis_name="c", subcore_axis_name="s")
    f = pl.kernel(
        functools.partial(_emb_body, n_chunks),
        out_type=jax.ShapeDtypeStruct((N, D), jnp.float32),
        mesh=mesh,
        scratch_types=[
            pltpu.VMEM((K,), jnp.int32),
            pltpu.VMEM((K, D), jnp.float32),
            pltpu.SemaphoreType.DMA,
        ],
        compiler_params=pltpu.CompilerParams(use_tc_tiling_on_sc=False),
    )
    out = f(xf, lut)
    return out.reshape(B, C, D)


# speedup vs baseline: 4.9348x; 1.0813x over previous
import functools
import math

import jax
import jax.numpy as jnp
from jax import lax
from jax.experimental import pallas as pl
from jax.experimental.pallas import tpu as pltpu
from jax.experimental.pallas import tpu_sc as plsc

D = 32
L = 16
SCALE = math.sqrt(D)
NC, NS = 2, 16
NW = NC * NS
K = 512
U = 8


def _emb_body(n, x_hbm, lut_hbm, out_hbm,
              idx_a, idx_b, rows_a, rows_b, ob_a, ob_b,
              sia, sib, sga, sgb, soa, sob):
    wid = lax.axis_index("s") * NC + lax.axis_index("c")
    base = wid * n * K

    idxs = (idx_a, idx_b)
    rows = (rows_a, rows_b)
    obs = (ob_a, ob_b)
    sis = (sia, sib)
    sgs = (sga, sgb)
    sos = (soa, sob)

    def off(ci):
        return base + ci * K

    pltpu.sync_copy(x_hbm.at[pl.ds(base, K)], idx_a)
    pltpu.async_copy(lut_hbm.at[idx_a], rows_a, sga)
    pltpu.async_copy(x_hbm.at[pl.ds(base + K, K)], idx_b, sib)

    def half(i, b):
        nb = 1 - b

        pltpu.make_async_copy(lut_hbm.at[idxs[b]], rows[b], sgs[b]).wait()

        @pl.when(i + 1 < n)
        def _():
            pltpu.make_async_copy(
                x_hbm.at[pl.ds(off(i + 1), K)], idxs[nb], sis[nb]).wait()
            pltpu.async_copy(lut_hbm.at[idxs[nb]], rows[nb], sgs[nb])

        @pl.when(i >= 2)
        def _():
            pltpu.make_async_copy(
                obs[b], out_hbm.at[pl.ds(off(i - 2), K)], sos[b]).wait()

        def row_body(r, c):
            r0 = r * U
            for u in range(U):
                for h in range(D // L):
                    sl = pl.ds(h * L, L)
                    obs[b][r0 + u, sl] = rows[b][r0 + u, sl] * SCALE
            return c

        lax.fori_loop(0, K // U, row_body, 0)

        pltpu.async_copy(obs[b], out_hbm.at[pl.ds(off(i), K)], sos[b])

        @pl.when(i + 2 < n)
        def _():
            pltpu.async_copy(x_hbm.at[pl.ds(off(i + 2), K)], idxs[b], sis[b])

    def outer(o, c):
        i = o * 2
        half(i, 0)
        half(i + 1, 1)
        return c

    lax.fori_loop(0, n // 2, outer, 0)

    pltpu.make_async_copy(ob_a, out_hbm.at[pl.ds(off(n - 2), K)], soa).wait()
    pltpu.make_async_copy(ob_b, out_hbm.at[pl.ds(off(n - 1), K)], sob).wait()


def kernel(x, lut):
    B, C = x.shape
    N = B * C
    assert N % (NW * K) == 0
    n_chunks = N // (NW * K)
    assert n_chunks % 2 == 0
    xf = x.reshape(N)

    mesh = plsc.VectorSubcoreMesh(core_axis_name="c", subcore_axis_name="s")
    f = pl.kernel(
        functools.partial(_emb_body, n_chunks),
        out_type=jax.ShapeDtypeStruct((N, D), jnp.float32),
        mesh=mesh,
        scratch_types=[
            pltpu.VMEM((K,), jnp.int32),
            pltpu.VMEM((K,), jnp.int32),
            pltpu.VMEM((K, D), jnp.float32),
            pltpu.VMEM((K, D), jnp.float32),
            pltpu.VMEM((K, D), jnp.float32),
            pltpu.VMEM((K, D), jnp.float32),
            pltpu.SemaphoreType.DMA,
            pltpu.SemaphoreType.DMA,
            pltpu.SemaphoreType.DMA,
            pltpu.SemaphoreType.DMA,
            pltpu.SemaphoreType.DMA,
            pltpu.SemaphoreType.DMA,
        ],
        compiler_params=pltpu.CompilerParams(use_tc_tiling_on_sc=False),
    )
    out = f(xf, lut)
    return out.reshape(B, C, D)


# speedup vs baseline: 7.0627x; 1.4312x over previous
import functools
import math

import jax
import jax.numpy as jnp
from jax import lax
from jax.experimental import pallas as pl
from jax.experimental.pallas import tpu as pltpu
from jax.experimental.pallas import tpu_sc as plsc

D = 32
L = 16
SCALE = math.sqrt(D)
NC, NS = 2, 16
NW = NC * NS
TE, EI = 4, 8
TB, BI = 4, 128
K = TB * BI
RU = 4


def _emb_body(n, xt_hbm, lut_hbm, out_hbm,
              idx_a, idx_b, rows_a, rows_b, ob_a, ob_b,
              sia, sib, sga, sgb, soa, sob):
    wid = lax.axis_index("s") * NC + lax.axis_index("c")
    tb0 = wid * TB
    b0 = wid * K

    idxs = (idx_a, idx_b)
    rows = (rows_a, rows_b)
    obs = (ob_a, ob_b)
    sis = (sia, sib)
    sgs = (sga, sgb)
    sos = (soa, sob)

    lane = lax.iota(jnp.int32, L)
    te_half = lane // EI
    ei_v = lane % EI

    pltpu.sync_copy(xt_hbm.at[0, pl.ds(b0, K)], idx_a)
    pltpu.async_copy(lut_hbm.at[idx_a], rows_a, sga)
    pltpu.async_copy(xt_hbm.at[1, pl.ds(b0, K)], idx_b, sib)

    def half(c, b):
        nb = 1 - b

        pltpu.make_async_copy(lut_hbm.at[idxs[b]], rows[b], sgs[b]).wait()

        @pl.when(c + 1 < n)
        def _():
            pltpu.make_async_copy(
                xt_hbm.at[c + 1, pl.ds(b0, K)], idxs[nb], sis[nb]).wait()
            pltpu.async_copy(lut_hbm.at[idxs[nb]], rows[nb], sgs[nb])

        @pl.when(c >= 2)
        def _():
            pltpu.make_async_copy(
                obs[b].at[:, :, :, pl.ds(0, BI)],
                out_hbm.at[c - 2, :, pl.ds(tb0, TB)], sos[b]).wait()

        def tloop(ro, carry):
            for u in range(RU):
                r = ro * RU + u
                tbl_v = jnp.zeros((L,), jnp.int32) + (r // BI)
                bi_v = jnp.zeros((L,), jnp.int32) + (r % BI)
                for h in range(D // L):
                    v = rows[b][r, pl.ds(h * L, L)]
                    plsc.store_scatter(
                        obs[b], [te_half + 2 * h, tbl_v, ei_v, bi_v],
                        v * SCALE)
            return carry

        lax.fori_loop(0, K // RU, tloop, 0)

        pltpu.async_copy(
            obs[b].at[:, :, :, pl.ds(0, BI)],
            out_hbm.at[c, :, pl.ds(tb0, TB)], sos[b])

        @pl.when(c + 2 < n)
        def _():
            pltpu.async_copy(xt_hbm.at[c + 2, pl.ds(b0, K)], idxs[b], sis[b])

    def outer(o, carry):
        c = o * 2
        half(c, 0)
        half(c + 1, 1)
        return carry

    lax.fori_loop(0, n // 2, outer, 0)

    pltpu.make_async_copy(
        ob_a.at[:, :, :, pl.ds(0, BI)],
        out_hbm.at[n - 2, :, pl.ds(tb0, TB)], soa).wait()
    pltpu.make_async_copy(
        ob_b.at[:, :, :, pl.ds(0, BI)],
        out_hbm.at[n - 1, :, pl.ds(tb0, TB)], sob).wait()


def kernel(x, lut):
    B, C = x.shape
    assert B == NW * K and C % 2 == 0 and lut.shape[1] == D
    xt = x.T

    mesh = plsc.VectorSubcoreMesh(core_axis_name="c", subcore_axis_name="s")
    f = pl.kernel(
        functools.partial(_emb_body, C),
        out_type=jax.ShapeDtypeStruct((C, TE, B // BI, EI, BI), jnp.float32),
        mesh=mesh,
        scratch_types=[
            pltpu.VMEM((K,), jnp.int32),
            pltpu.VMEM((K,), jnp.int32),
            pltpu.VMEM((K, D), jnp.float32),
            pltpu.VMEM((K, D), jnp.float32),
            pltpu.VMEM((TE, TB, EI, BI + 1), jnp.float32),
            pltpu.VMEM((TE, TB, EI, BI + 1), jnp.float32),
            pltpu.SemaphoreType.DMA,
            pltpu.SemaphoreType.DMA,
            pltpu.SemaphoreType.DMA,
            pltpu.SemaphoreType.DMA,
            pltpu.SemaphoreType.DMA,
            pltpu.SemaphoreType.DMA,
        ],
        compiler_params=pltpu.CompilerParams(
            use_tc_tiling_on_sc=False, needs_layout_passes=False),
    )
    o5 = f(xt, lut)
    return o5.transpose((2, 4, 0, 1, 3)).reshape(B, C, D)
